# baseline (speedup 1.0000x reference)
import functools
import numpy as np
import jax
import jax.numpy as jnp
from jax import lax
from jax.experimental import pallas as pl
from jax.experimental.pallas import tpu as pltpu
from jax.experimental.pallas import tpu_sc as plsc

N = 10000
E = 320000
D = 128
DE = 16

NC = 2
NS = 16
NW = NC * NS
EPW = E // NW
K = 40
NCHUNK = EPW // K
NPH = 2
PCH = NCHUNK // NPH
CH = 624
TAIL = N - NS * CH

_Q = np.empty((D,), np.int32)
for _j in range(D // 32):
    for _i in range(16):
        _Q[16 * _j + _i] = 32 * _j + _i
        _Q[64 + 16 * _j + _i] = 32 * _j + 16 + _i


def _dot(a, b):
    return jnp.dot(a, b, preferred_element_type=jnp.float32,
                   precision=lax.Precision.HIGHEST)


def _bdot(a, b):
    return jnp.dot(a.astype(jnp.bfloat16), b.astype(jnp.bfloat16),
                   preferred_element_type=jnp.float32)


def _prep_node_body(x_ref, wa_ref, wb_ref, ws_ref, a_ref, b_ref, s_ref):
    x = x_ref[...]
    a_ref[...] = _bdot(x, wa_ref[...])
    b_ref[...] = _bdot(x, wb_ref[...])
    s_ref[...] = _dot(x, ws_ref[...])


def _prep_node(x, wa, wb, ws):
    bn = 1000
    grid = (N // bn,)
    out = [jax.ShapeDtypeStruct((N, D), jnp.float32)] * 3
    return pl.pallas_call(
        _prep_node_body,
        grid=grid,
        in_specs=[
            pl.BlockSpec((bn, D), lambda i: (i, 0)),
            pl.BlockSpec((D, D), lambda i: (0, 0)),
            pl.BlockSpec((D, D), lambda i: (0, 0)),
            pl.BlockSpec((D, D), lambda i: (0, 0)),
        ],
        out_specs=[pl.BlockSpec((bn, D), lambda i: (i, 0))] * 3,
        out_shape=out,
    )(x, wa, wb, ws)


def _prep_edge_body(ea_ref, w_ref, b_ref, c_ref):
    c = _bdot(ea_ref[...], w_ref[...]) + b_ref[...]
    ci = lax.bitcast_convert_type(c, jnp.int32)
    lo = ci[:, :D // 2]
    hi = ci[:, D // 2:]
    lo = (lo + 32768 + ((lo >> 16) & 1)) >> 16
    lo = lo & 65535
    hi = (hi + 32768 + ((hi >> 16) & 1)) & (-65536)
    c_ref[...] = lo | hi


def _prep_edge(ea, wa, b):
    be = 4000
    grid = (E // be,)
    return pl.pallas_call(
        _prep_edge_body,
        grid=grid,
        in_specs=[
            pl.BlockSpec((be, DE), lambda i: (i, 0)),
            pl.BlockSpec((DE, D), lambda i: (0, 0)),
            pl.BlockSpec((1, D), lambda i: (0, 0)),
        ],
        out_specs=pl.BlockSpec((be, D // 2), lambda i: (i, 0)),
        out_shape=jax.ShapeDtypeStruct((E, D // 2), jnp.int32),
    )(ea, wa, b.reshape(1, D))


def _mlp1_body(x_ref, p_ref, wt_ref, wb_ref, bias_ref, ws1_ref, wd1_ref,
               h_ref, a1_ref, b1_ref):
    agg = p_ref[0] + p_ref[1]
    h = _dot(x_ref[...], wt_ref[...]) + _dot(agg, wb_ref[...])
    h = jnp.maximum(h + bias_ref[...], 0.0)
    h_ref[...] = h
    a1_ref[...] = _bdot(h, ws1_ref[...])
    b1_ref[...] = _bdot(h, wd1_ref[...])


def _mlp1(x, p, wt, wb, bias, ws1, wd1):
    bn = 1000
    grid = (N // bn,)
    out = [jax.ShapeDtypeStruct((N, D), jnp.float32)] * 3
    return pl.pallas_call(
        _mlp1_body,
        grid=grid,
        in_specs=[
            pl.BlockSpec((bn, D), lambda i: (i, 0)),
            pl.BlockSpec((2, bn, D), lambda i: (0, i, 0)),
            pl.BlockSpec((D, D), lambda i: (0, 0)),
            pl.BlockSpec((D, D), lambda i: (0, 0)),
            pl.BlockSpec((1, D), lambda i: (0, 0)),
            pl.BlockSpec((D, D), lambda i: (0, 0)),
            pl.BlockSpec((D, D), lambda i: (0, 0)),
        ],
        out_specs=[pl.BlockSpec((bn, D), lambda i: (i, 0))] * 3,
        out_shape=out,
    )(x, p, wt, wb, bias.reshape(1, D), ws1, wd1)


def _mlp2_body(h_ref, p_ref, skip_ref, wt_ref, wb_ref, bias_ref, o_ref):
    agg = p_ref[0] + p_ref[1]
    o = _dot(h_ref[...], wt_ref[...]) + _dot(agg, wb_ref[...])
    o_ref[...] = jnp.maximum(o + bias_ref[...], 0.0) + skip_ref[...]


def _mlp2(h, p, skip, wt, wb, bias):
    bn = 1000
    grid = (N // bn,)
    return pl.pallas_call(
        _mlp2_body,
        grid=grid,
        in_specs=[
            pl.BlockSpec((bn, D), lambda i: (i, 0)),
            pl.BlockSpec((2, bn, D), lambda i: (0, i, 0)),
            pl.BlockSpec((bn, D), lambda i: (i, 0)),
            pl.BlockSpec((D, D), lambda i: (0, 0)),
            pl.BlockSpec((D, D), lambda i: (0, 0)),
            pl.BlockSpec((1, D), lambda i: (0, 0)),
        ],
        out_specs=pl.BlockSpec((bn, D), lambda i: (i, 0)),
        out_shape=jax.ShapeDtypeStruct((N, D), jnp.float32),
    )(h, p, skip, wt, wb, bias.reshape(1, D))


def _edge_sc_body(a_hbm, b_hbm, c_hbm, src_hbm, dst_hbm, out_hbm,
                  dstv, sv0, sv1, av0, bv0, cv0, av1, bv1, cv1, agg_sh,
                  sa0, sb0, sc0, sa1, sb1, sc1, ss0, ss1):
    cid = lax.axis_index("c")
    sid = lax.axis_index("s")
    wid = sid * NC + cid

    gbufs = ((av0, bv0, cv0), (av1, bv1, cv1))
    gsems = ((sa0, sb0, sc0), (sa1, sb1, sc1))
    sbufs = (sv0, sv1)
    ssems = (ss0, ss1)

    zero16 = jnp.zeros((16,), jnp.float32)

    def zrow(r, _):
        for j in range(D // 16):
            av0[r, pl.ds(j * 16, 16)] = zero16
        return 0

    lax.fori_loop(0, K, zrow, 0)
    nz = CH // K
    rem = CH - nz * K

    def zcopy(i, _):
        pltpu.sync_copy(av0, agg_sh.at[pl.ds(sid * CH + i * K, K)])
        return 0

    lax.fori_loop(0, nz, zcopy, 0)
    if rem > 0:
        pltpu.sync_copy(av0.at[pl.ds(0, rem)],
                        agg_sh.at[pl.ds(sid * CH + nz * K, rem)])

    @pl.when(sid == 0)
    def _():
        pltpu.sync_copy(av0.at[pl.ds(0, TAIL)],
                        agg_sh.at[pl.ds(NS * CH, TAIL)])

    plsc.subcore_barrier()

    base_e = wid * EPW

    for ph in range(NPH):
        pltpu.sync_copy(dst_hbm.at[wid, ph], dstv)
        pbase = ph * PCH

        def fetch_src(g, b):
            pltpu.async_copy(src_hbm.at[wid, pbase + g], sbufs[b], ssems[b])

        H1, H2 = 24, 16

        def issue_gathers(g, b):
            av, bv, cv = gbufs[b]
            sa, sb, sc = gsems[b]
            pltpu.make_async_copy(src_hbm.at[wid, pbase + g], sbufs[b],
                                  ssems[b]).wait()
            pltpu.async_copy(a_hbm.at[sbufs[b].at[pl.ds(0, H1)]],
                             av.at[pl.ds(0, H1)], sa)
            pltpu.async_copy(b_hbm.at[dstv.at[g, pl.ds(0, H1)]],
                             bv.at[pl.ds(0, H1)], sb)
            pltpu.async_copy(a_hbm.at[sbufs[b].at[pl.ds(H1, H2)]],
                             av.at[pl.ds(H1, H2)], sa)
            pltpu.async_copy(b_hbm.at[dstv.at[g, pl.ds(H1, H2)]],
                             bv.at[pl.ds(H1, H2)], sb)
            pltpu.async_copy(
                c_hbm.at[pl.ds(base_e + (pbase + g) * K, K)], cv, sc)

        def finish(g, b):
            av, bv, cv = gbufs[b]
            sa, sb, sc = gsems[b]
            pltpu.make_async_copy(a_hbm.at[sbufs[b].at[pl.ds(0, H1)]],
                                  av.at[pl.ds(0, H1)], sa).wait()
            pltpu.make_async_copy(b_hbm.at[dstv.at[g, pl.ds(0, H1)]],
                                  bv.at[pl.ds(0, H1)], sb).wait()
            pltpu.make_async_copy(a_hbm.at[sbufs[b].at[pl.ds(H1, H2)]],
                                  av.at[pl.ds(H1, H2)], sa).wait()
            pltpu.make_async_copy(b_hbm.at[dstv.at[g, pl.ds(H1, H2)]],
                                  bv.at[pl.ds(H1, H2)], sb).wait()
            pltpu.make_async_copy(
                c_hbm.at[pl.ds(base_e + (pbase + g) * K, K)], cv, sc).wait()

            shv = jnp.full((16,), 16, jnp.int32)
            mkv = jnp.full((16,), -65536, jnp.int32)
            bc = lambda v: lax.bitcast_convert_type(v, jnp.float32)

            def row(r, _):
                for j in range(D // 32):
                    cw = cv[r, pl.ds(16 * j, 16)]
                    clo = bc(lax.shift_left(cw, shv))
                    chi = bc(lax.bitwise_and(cw, mkv))
                    slo = pl.ds(32 * j, 16)
                    shi = pl.ds(32 * j + 16, 16)
                    av[r, slo] = jnp.maximum(av[r, slo] + bv[r, slo] + clo,
                                             0.0)
                    av[r, shi] = jnp.maximum(av[r, shi] + bv[r, shi] + chi,
                                             0.0)
                return 0

            lax.fori_loop(0, K, row, 0)
            pltpu.sync_copy(av, agg_sh.at[dstv.at[g]], add=True)

        fetch_src(0, 0)
        fetch_src(1, 1)
        issue_gathers(0, 0)

        def pair(p, _):
            g = 2 * p
            issue_gathers(g + 1, 1)
            fetch_src(g + 2, 0)
            finish(g, 0)
            issue_gathers(g + 2, 0)

            @pl.when(g + 3 < PCH)
            def _():
                fetch_src(g + 3, 1)

            finish(g + 1, 1)
            return 0

        lax.fori_loop(0, (PCH - 1) // 2, pair, 0)
        finish(PCH - 1, 0)

    plsc.subcore_barrier()

    r0 = sid * CH
    pltpu.sync_copy(agg_sh.at[pl.ds(r0, CH)], out_hbm.at[cid, pl.ds(r0, CH)])

    @pl.when(sid == 0)
    def _():
        pltpu.sync_copy(agg_sh.at[pl.ds(NS * CH, TAIL)],
                        out_hbm.at[cid, pl.ds(NS * CH, TAIL)])


@functools.cache
def _build_edge_pass():
    return pl.kernel(
        _edge_sc_body,
        out_type=jax.ShapeDtypeStruct((NC, N, D), jnp.float32),
        mesh=plsc.VectorSubcoreMesh(core_axis_name="c", subcore_axis_name="s",
                                    num_cores=NC, num_subcores=NS),
        scratch_types=[
            pltpu.VMEM((PCH, K), jnp.int32),
            pltpu.VMEM((K,), jnp.int32),
            pltpu.VMEM((K,), jnp.int32),
            pltpu.VMEM((K, D), jnp.float32),
            pltpu.VMEM((K, D), jnp.float32),
            pltpu.VMEM((K, D // 2), jnp.int32),
            pltpu.VMEM((K, D), jnp.float32),
            pltpu.VMEM((K, D), jnp.float32),
            pltpu.VMEM((K, D // 2), jnp.int32),
            pltpu.VMEM_SHARED((N, D), jnp.float32),
            pltpu.SemaphoreType.DMA,
            pltpu.SemaphoreType.DMA,
            pltpu.SemaphoreType.DMA,
            pltpu.SemaphoreType.DMA,
            pltpu.SemaphoreType.DMA,
            pltpu.SemaphoreType.DMA,
            pltpu.SemaphoreType.DMA,
            pltpu.SemaphoreType.DMA,
        ],
    )


def _edge_pass(a, b, c, src, dst):
    return _build_edge_pass()(a, b, c, src, dst)


def kernel(node_feat, node_attr, edge_index, edge_attr, batch_index,
           num_sampled_nodes_per_hop, num_sampled_edges_per_hop,
           W_e0, b_e0, W_n0, b_n0, W_e1, b_e1, W_n1, b_n1, W_skip):
    src = edge_index[0].reshape(NW, NCHUNK, K)
    dst = edge_index[1].reshape(NW, NPH, PCH, K)

    Ws0, Wd0, Wr0, Wa0 = (W_e0[:D], W_e0[D:2 * D], W_e0[2 * D:3 * D],
                          W_e0[3 * D:])
    Wsrc0 = Ws0 - Wr0
    Wdst0 = Wd0 + Wr0
    Ws1, Wd1, Wa1 = W_e1[:D], W_e1[D:2 * D], W_e1[2 * D:]

    A0, B0, S = _prep_node(node_feat, Wsrc0, Wdst0, W_skip)
    C0 = _prep_edge(edge_attr, Wa0[:, _Q], b_e0[_Q])

    P0 = _edge_pass(A0, B0, C0, src, dst)
    C1 = _prep_edge(edge_attr, Wa1[:, _Q], b_e1[_Q])
    h1, A1, B1 = _mlp1(node_feat, P0, W_n0[:D], W_n0[D:], b_n0, Ws1, Wd1)

    P1 = _edge_pass(A1, B1, C1, src, dst)
    out = _mlp2(h1, P1, S, W_n1[:D], W_n1[D:], b_n1)

    return (out, node_attr, edge_index, edge_attr)


# speedup vs baseline: 1.0221x; 1.0221x over previous
import functools
import numpy as np
import jax
import jax.numpy as jnp
from jax import lax
from jax.experimental import pallas as pl
from jax.experimental.pallas import tpu as pltpu
from jax.experimental.pallas import tpu_sc as plsc

N = 10000
E = 320000
D = 128
DE = 16

NC = 2
NS = 16
NW = NC * NS
EPW = E // NW
K = 40
NCHUNK = EPW // K
NPH = 2
PCH = NCHUNK // NPH
CH = 624
TAIL = N - NS * CH

_Q = np.empty((D,), np.int32)
for _j in range(D // 32):
    for _i in range(16):
        _Q[16 * _j + _i] = 32 * _j + _i
        _Q[64 + 16 * _j + _i] = 32 * _j + 16 + _i


def _dot(a, b):
    return jnp.dot(a, b, preferred_element_type=jnp.float32,
                   precision=lax.Precision.HIGHEST)


def _bdot(a, b):
    return jnp.dot(a.astype(jnp.bfloat16), b.astype(jnp.bfloat16),
                   preferred_element_type=jnp.float32)


def _prep_node_body(x_ref, wa_ref, wb_ref, ws_ref, a_ref, b_ref, s_ref):
    x = x_ref[...]
    a_ref[...] = _bdot(x, wa_ref[...])
    b_ref[...] = _bdot(x, wb_ref[...])
    s_ref[...] = _dot(x, ws_ref[...])


def _prep_node(x, wa, wb, ws):
    bn = 1000
    grid = (N // bn,)
    out = [jax.ShapeDtypeStruct((N, D), jnp.float32)] * 3
    return pl.pallas_call(
        _prep_node_body,
        grid=grid,
        in_specs=[
            pl.BlockSpec((bn, D), lambda i: (i, 0)),
            pl.BlockSpec((D, D), lambda i: (0, 0)),
            pl.BlockSpec((D, D), lambda i: (0, 0)),
            pl.BlockSpec((D, D), lambda i: (0, 0)),
        ],
        out_specs=[pl.BlockSpec((bn, D), lambda i: (i, 0))] * 3,
        out_shape=out,
    )(x, wa, wb, ws)


def _prep_edge_body(ea_ref, w_ref, b_ref, c_ref):
    c = _bdot(ea_ref[...], w_ref[...]) + b_ref[...]
    ci = lax.bitcast_convert_type(c, jnp.int32)
    lo = ci[:, :D // 2]
    hi = ci[:, D // 2:]
    lo = (lo + 32768 + ((lo >> 16) & 1)) >> 16
    lo = lo & 65535
    hi = (hi + 32768 + ((hi >> 16) & 1)) & (-65536)
    c_ref[...] = lo | hi


def _prep_edge(ea, wa, b):
    be = 8000
    grid = (E // be,)
    return pl.pallas_call(
        _prep_edge_body,
        grid=grid,
        in_specs=[
            pl.BlockSpec((be, DE), lambda i: (i, 0)),
            pl.BlockSpec((DE, D), lambda i: (0, 0)),
            pl.BlockSpec((1, D), lambda i: (0, 0)),
        ],
        out_specs=pl.BlockSpec((be, D // 2), lambda i: (i, 0)),
        out_shape=jax.ShapeDtypeStruct((E, D // 2), jnp.int32),
    )(ea, wa, b.reshape(1, D))


def _mlp1_body(x_ref, p_ref, wt_ref, wb_ref, bias_ref, ws1_ref, wd1_ref,
               h_ref, a1_ref, b1_ref):
    agg = p_ref[0] + p_ref[1]
    h = _dot(x_ref[...], wt_ref[...]) + _dot(agg, wb_ref[...])
    h = jnp.maximum(h + bias_ref[...], 0.0)
    h_ref[...] = h
    a1_ref[...] = _bdot(h, ws1_ref[...])
    b1_ref[...] = _bdot(h, wd1_ref[...])


def _mlp1(x, p, wt, wb, bias, ws1, wd1):
    bn = 1000
    grid = (N // bn,)
    out = [jax.ShapeDtypeStruct((N, D), jnp.float32)] * 3
    return pl.pallas_call(
        _mlp1_body,
        grid=grid,
        in_specs=[
            pl.BlockSpec((bn, D), lambda i: (i, 0)),
            pl.BlockSpec((2, bn, D), lambda i: (0, i, 0)),
            pl.BlockSpec((D, D), lambda i: (0, 0)),
            pl.BlockSpec((D, D), lambda i: (0, 0)),
            pl.BlockSpec((1, D), lambda i: (0, 0)),
            pl.BlockSpec((D, D), lambda i: (0, 0)),
            pl.BlockSpec((D, D), lambda i: (0, 0)),
        ],
        out_specs=[pl.BlockSpec((bn, D), lambda i: (i, 0))] * 3,
        out_shape=out,
    )(x, p, wt, wb, bias.reshape(1, D), ws1, wd1)


def _mlp2_body(h_ref, p_ref, skip_ref, wt_ref, wb_ref, bias_ref, o_ref):
    agg = p_ref[0] + p_ref[1]
    o = _dot(h_ref[...], wt_ref[...]) + _dot(agg, wb_ref[...])
    o_ref[...] = jnp.maximum(o + bias_ref[...], 0.0) + skip_ref[...]


def _mlp2(h, p, skip, wt, wb, bias):
    bn = 1000
    grid = (N // bn,)
    return pl.pallas_call(
        _mlp2_body,
        grid=grid,
        in_specs=[
            pl.BlockSpec((bn, D), lambda i: (i, 0)),
            pl.BlockSpec((2, bn, D), lambda i: (0, i, 0)),
            pl.BlockSpec((bn, D), lambda i: (i, 0)),
            pl.BlockSpec((D, D), lambda i: (0, 0)),
            pl.BlockSpec((D, D), lambda i: (0, 0)),
            pl.BlockSpec((1, D), lambda i: (0, 0)),
        ],
        out_specs=pl.BlockSpec((bn, D), lambda i: (i, 0)),
        out_shape=jax.ShapeDtypeStruct((N, D), jnp.float32),
    )(h, p, skip, wt, wb, bias.reshape(1, D))


def _edge_sc_body(a_hbm, b_hbm, c_hbm, src_hbm, dst_hbm, out_hbm,
                  dstv, sv0, sv1, av0, bv0, cv0, av1, bv1, cv1, agg_sh,
                  sa0, sb0, sc0, sa1, sb1, sc1, ss0, ss1):
    cid = lax.axis_index("c")
    sid = lax.axis_index("s")
    wid = sid * NC + cid

    gbufs = ((av0, bv0, cv0), (av1, bv1, cv1))
    gsems = ((sa0, sb0, sc0), (sa1, sb1, sc1))
    sbufs = (sv0, sv1)
    ssems = (ss0, ss1)

    zero16 = jnp.zeros((16,), jnp.float32)

    def zrow(r, _):
        for j in range(D // 16):
            av0[r, pl.ds(j * 16, 16)] = zero16
        return 0

    lax.fori_loop(0, K, zrow, 0)
    nz = CH // K
    rem = CH - nz * K

    def zcopy(i, _):
        pltpu.sync_copy(av0, agg_sh.at[pl.ds(sid * CH + i * K, K)])
        return 0

    lax.fori_loop(0, nz, zcopy, 0)
    if rem > 0:
        pltpu.sync_copy(av0.at[pl.ds(0, rem)],
                        agg_sh.at[pl.ds(sid * CH + nz * K, rem)])

    @pl.when(sid == 0)
    def _():
        pltpu.sync_copy(av0.at[pl.ds(0, TAIL)],
                        agg_sh.at[pl.ds(NS * CH, TAIL)])

    plsc.subcore_barrier()

    base_e = wid * EPW

    for ph in range(NPH):
        pltpu.sync_copy(dst_hbm.at[wid, ph], dstv)
        pbase = ph * PCH

        def fetch_src(g, b):
            pltpu.async_copy(src_hbm.at[wid, pbase + g], sbufs[b], ssems[b])

        def issue_gathers(g, b):
            av, bv, cv = gbufs[b]
            sa, sb, sc = gsems[b]
            pltpu.make_async_copy(src_hbm.at[wid, pbase + g], sbufs[b],
                                  ssems[b]).wait()
            pltpu.async_copy(a_hbm.at[sbufs[b]], av, sa)
            pltpu.async_copy(b_hbm.at[dstv.at[g]], bv, sb)
            pltpu.async_copy(
                c_hbm.at[pl.ds(base_e + (pbase + g) * K, K)], cv, sc)

        def finish(g, b):
            av, bv, cv = gbufs[b]
            sa, sb, sc = gsems[b]
            pltpu.make_async_copy(a_hbm.at[sbufs[b]], av, sa).wait()
            pltpu.make_async_copy(b_hbm.at[dstv.at[g]], bv, sb).wait()
            pltpu.make_async_copy(
                c_hbm.at[pl.ds(base_e + (pbase + g) * K, K)], cv, sc).wait()

            shv = jnp.full((16,), 16, jnp.int32)
            mkv = jnp.full((16,), -65536, jnp.int32)
            bc = lambda v: lax.bitcast_convert_type(v, jnp.float32)

            def row(r, _):
                for j in range(D // 32):
                    cw = cv[r, pl.ds(16 * j, 16)]
                    clo = bc(lax.shift_left(cw, shv))
                    chi = bc(lax.bitwise_and(cw, mkv))
                    slo = pl.ds(32 * j, 16)
                    shi = pl.ds(32 * j + 16, 16)
                    av[r, slo] = jnp.maximum(av[r, slo] + bv[r, slo] + clo,
                                             0.0)
                    av[r, shi] = jnp.maximum(av[r, shi] + bv[r, shi] + chi,
                                             0.0)
                return 0

            lax.fori_loop(0, K, row, 0)
            pltpu.sync_copy(av, agg_sh.at[dstv.at[g]], add=True)

        fetch_src(0, 0)
        fetch_src(1, 1)
        issue_gathers(0, 0)

        def pair(p, _):
            g = 2 * p
            issue_gathers(g + 1, 1)
            fetch_src(g + 2, 0)
            finish(g, 0)
            issue_gathers(g + 2, 0)

            @pl.when(g + 3 < PCH)
            def _():
                fetch_src(g + 3, 1)

            finish(g + 1, 1)
            return 0

        lax.fori_loop(0, (PCH - 1) // 2, pair, 0)
        finish(PCH - 1, 0)

    plsc.subcore_barrier()

    r0 = sid * CH
    pltpu.sync_copy(agg_sh.at[pl.ds(r0, CH)], out_hbm.at[cid, pl.ds(r0, CH)])

    @pl.when(sid == 0)
    def _():
        pltpu.sync_copy(agg_sh.at[pl.ds(NS * CH, TAIL)],
                        out_hbm.at[cid, pl.ds(NS * CH, TAIL)])


@functools.cache
def _build_edge_pass():
    return pl.kernel(
        _edge_sc_body,
        out_type=jax.ShapeDtypeStruct((NC, N, D), jnp.float32),
        mesh=plsc.VectorSubcoreMesh(core_axis_name="c", subcore_axis_name="s",
                                    num_cores=NC, num_subcores=NS),
        scratch_types=[
            pltpu.VMEM((PCH, K), jnp.int32),
            pltpu.VMEM((K,), jnp.int32),
            pltpu.VMEM((K,), jnp.int32),
            pltpu.VMEM((K, D), jnp.float32),
            pltpu.VMEM((K, D), jnp.float32),
            pltpu.VMEM((K, D // 2), jnp.int32),
            pltpu.VMEM((K, D), jnp.float32),
            pltpu.VMEM((K, D), jnp.float32),
            pltpu.VMEM((K, D // 2), jnp.int32),
            pltpu.VMEM_SHARED((N, D), jnp.float32),
            pltpu.SemaphoreType.DMA,
            pltpu.SemaphoreType.DMA,
            pltpu.SemaphoreType.DMA,
            pltpu.SemaphoreType.DMA,
            pltpu.SemaphoreType.DMA,
            pltpu.SemaphoreType.DMA,
            pltpu.SemaphoreType.DMA,
            pltpu.SemaphoreType.DMA,
        ],
    )


def _edge_pass(a, b, c, src, dst):
    return _build_edge_pass()(a, b, c, src, dst)


def kernel(node_feat, node_attr, edge_index, edge_attr, batch_index,
           num_sampled_nodes_per_hop, num_sampled_edges_per_hop,
           W_e0, b_e0, W_n0, b_n0, W_e1, b_e1, W_n1, b_n1, W_skip):
    src = edge_index[0].reshape(NW, NCHUNK, K)
    dst = edge_index[1].reshape(NW, NPH, PCH, K)

    Ws0, Wd0, Wr0, Wa0 = (W_e0[:D], W_e0[D:2 * D], W_e0[2 * D:3 * D],
                          W_e0[3 * D:])
    Wsrc0 = Ws0 - Wr0
    Wdst0 = Wd0 + Wr0
    Ws1, Wd1, Wa1 = W_e1[:D], W_e1[D:2 * D], W_e1[2 * D:]

    A0, B0, S = _prep_node(node_feat, Wsrc0, Wdst0, W_skip)
    C0 = _prep_edge(edge_attr, Wa0[:, _Q], b_e0[_Q])

    P0 = _edge_pass(A0, B0, C0, src, dst)
    C1 = _prep_edge(edge_attr, Wa1[:, _Q], b_e1[_Q])
    h1, A1, B1 = _mlp1(node_feat, P0, W_n0[:D], W_n0[D:], b_n0, Ws1, Wd1)

    P1 = _edge_pass(A1, B1, C1, src, dst)
    out = _mlp2(h1, P1, S, W_n1[:D], W_n1[D:], b_n1)

    return (out, node_attr, edge_index, edge_attr)


# speedup vs baseline: 1.0365x; 1.0142x over previous
import functools
import numpy as np
import jax
import jax.numpy as jnp
from jax import lax
from jax.experimental import pallas as pl
from jax.experimental.pallas import tpu as pltpu
from jax.experimental.pallas import tpu_sc as plsc

N = 10000
E = 320000
D = 128
DE = 16

NC = 2
NS = 16
NW = NC * NS
EPW = E // NW
K = 40
NCHUNK = EPW // K
NPH = 2
PCH = NCHUNK // NPH
CH = 624
TAIL = N - NS * CH

_Q = np.empty((D,), np.int32)
for _j in range(D // 32):
    for _i in range(16):
        _Q[16 * _j + _i] = 32 * _j + _i
        _Q[64 + 16 * _j + _i] = 32 * _j + 16 + _i


def _dot(a, b):
    return jnp.dot(a, b, preferred_element_type=jnp.float32,
                   precision=lax.Precision.HIGHEST)


def _bdot(a, b):
    return jnp.dot(a.astype(jnp.bfloat16), b.astype(jnp.bfloat16),
                   preferred_element_type=jnp.float32)


def _prep_node_body(x_ref, wa_ref, wb_ref, ws_ref, a_ref, b_ref, s_ref):
    x = x_ref[...]
    a_ref[...] = _bdot(x, wa_ref[...])
    b_ref[...] = _bdot(x, wb_ref[...])
    s_ref[...] = _dot(x, ws_ref[...])


def _prep_node(x, wa, wb, ws):
    bn = 2000
    grid = (N // bn,)
    out = [jax.ShapeDtypeStruct((N, D), jnp.float32)] * 3
    return pl.pallas_call(
        _prep_node_body,
        grid=grid,
        in_specs=[
            pl.BlockSpec((bn, D), lambda i: (i, 0)),
            pl.BlockSpec((D, D), lambda i: (0, 0)),
            pl.BlockSpec((D, D), lambda i: (0, 0)),
            pl.BlockSpec((D, D), lambda i: (0, 0)),
        ],
        out_specs=[pl.BlockSpec((bn, D), lambda i: (i, 0))] * 3,
        out_shape=out,
    )(x, wa, wb, ws)


def _prep_edge_body(ea_ref, w_ref, b_ref, c_ref):
    c = _bdot(ea_ref[...], w_ref[...]) + b_ref[...]
    ci = lax.bitcast_convert_type(c, jnp.int32)
    lo = ci[:, :D // 2]
    hi = ci[:, D // 2:]
    lo = (lo + 32768 + ((lo >> 16) & 1)) >> 16
    lo = lo & 65535
    hi = (hi + 32768 + ((hi >> 16) & 1)) & (-65536)
    c_ref[...] = lo | hi


def _prep_edge(ea, wa, b):
    be = 8000
    grid = (E // be,)
    return pl.pallas_call(
        _prep_edge_body,
        grid=grid,
        in_specs=[
            pl.BlockSpec((be, DE), lambda i: (i, 0)),
            pl.BlockSpec((DE, D), lambda i: (0, 0)),
            pl.BlockSpec((1, D), lambda i: (0, 0)),
        ],
        out_specs=pl.BlockSpec((be, D // 2), lambda i: (i, 0)),
        out_shape=jax.ShapeDtypeStruct((E, D // 2), jnp.int32),
    )(ea, wa, b.reshape(1, D))


def _mlp1_body(x_ref, p_ref, wt_ref, wb_ref, bias_ref, ws1_ref, wd1_ref,
               h_ref, a1_ref, b1_ref):
    agg = p_ref[0] + p_ref[1]
    h = _dot(x_ref[...], wt_ref[...]) + _dot(agg, wb_ref[...])
    h = jnp.maximum(h + bias_ref[...], 0.0)
    h_ref[...] = h
    a1_ref[...] = _bdot(h, ws1_ref[...])
    b1_ref[...] = _bdot(h, wd1_ref[...])


def _mlp1(x, p, wt, wb, bias, ws1, wd1):
    bn = 2000
    grid = (N // bn,)
    out = [jax.ShapeDtypeStruct((N, D), jnp.float32)] * 3
    return pl.pallas_call(
        _mlp1_body,
        grid=grid,
        in_specs=[
            pl.BlockSpec((bn, D), lambda i: (i, 0)),
            pl.BlockSpec((2, bn, D), lambda i: (0, i, 0)),
            pl.BlockSpec((D, D), lambda i: (0, 0)),
            pl.BlockSpec((D, D), lambda i: (0, 0)),
            pl.BlockSpec((1, D), lambda i: (0, 0)),
            pl.BlockSpec((D, D), lambda i: (0, 0)),
            pl.BlockSpec((D, D), lambda i: (0, 0)),
        ],
        out_specs=[pl.BlockSpec((bn, D), lambda i: (i, 0))] * 3,
        out_shape=out,
    )(x, p, wt, wb, bias.reshape(1, D), ws1, wd1)


def _mlp2_body(h_ref, p_ref, skip_ref, wt_ref, wb_ref, bias_ref, o_ref):
    agg = p_ref[0] + p_ref[1]
    o = _dot(h_ref[...], wt_ref[...]) + _dot(agg, wb_ref[...])
    o_ref[...] = jnp.maximum(o + bias_ref[...], 0.0) + skip_ref[...]


def _mlp2(h, p, skip, wt, wb, bias):
    bn = 2000
    grid = (N // bn,)
    return pl.pallas_call(
        _mlp2_body,
        grid=grid,
        in_specs=[
            pl.BlockSpec((bn, D), lambda i: (i, 0)),
            pl.BlockSpec((2, bn, D), lambda i: (0, i, 0)),
            pl.BlockSpec((bn, D), lambda i: (i, 0)),
            pl.BlockSpec((D, D), lambda i: (0, 0)),
            pl.BlockSpec((D, D), lambda i: (0, 0)),
            pl.BlockSpec((1, D), lambda i: (0, 0)),
        ],
        out_specs=pl.BlockSpec((bn, D), lambda i: (i, 0)),
        out_shape=jax.ShapeDtypeStruct((N, D), jnp.float32),
    )(h, p, skip, wt, wb, bias.reshape(1, D))


def _edge_sc_body(a_hbm, b_hbm, c_hbm, src_hbm, dst_hbm, out_hbm,
                  dstv, sv0, sv1, av0, bv0, cv0, av1, bv1, cv1, agg_sh,
                  sa0, sb0, sc0, sa1, sb1, sc1, ss0, ss1):
    cid = lax.axis_index("c")
    sid = lax.axis_index("s")
    wid = sid * NC + cid

    gbufs = ((av0, bv0, cv0), (av1, bv1, cv1))
    gsems = ((sa0, sb0, sc0), (sa1, sb1, sc1))
    sbufs = (sv0, sv1)
    ssems = (ss0, ss1)

    zero16 = jnp.zeros((16,), jnp.float32)

    def zrow(r, _):
        for j in range(D // 16):
            av0[r, pl.ds(j * 16, 16)] = zero16
        return 0

    lax.fori_loop(0, K, zrow, 0)
    nz = CH // K
    rem = CH - nz * K

    def zcopy(i, _):
        pltpu.sync_copy(av0, agg_sh.at[pl.ds(sid * CH + i * K, K)])
        return 0

    lax.fori_loop(0, nz, zcopy, 0)
    if rem > 0:
        pltpu.sync_copy(av0.at[pl.ds(0, rem)],
                        agg_sh.at[pl.ds(sid * CH + nz * K, rem)])

    @pl.when(sid == 0)
    def _():
        pltpu.sync_copy(av0.at[pl.ds(0, TAIL)],
                        agg_sh.at[pl.ds(NS * CH, TAIL)])

    plsc.subcore_barrier()

    base_e = wid * EPW

    for ph in range(NPH):
        pltpu.sync_copy(dst_hbm.at[wid, ph], dstv)
        pbase = ph * PCH

        def fetch_src(g, b):
            pltpu.async_copy(src_hbm.at[wid, pbase + g], sbufs[b], ssems[b])

        def issue_gathers(g, b):
            av, bv, cv = gbufs[b]
            sa, sb, sc = gsems[b]
            pltpu.make_async_copy(src_hbm.at[wid, pbase + g], sbufs[b],
                                  ssems[b]).wait()
            pltpu.async_copy(a_hbm.at[sbufs[b]], av, sa)
            pltpu.async_copy(b_hbm.at[dstv.at[g]], bv, sb)
            pltpu.async_copy(
                c_hbm.at[pl.ds(base_e + (pbase + g) * K, K)], cv, sc)

        def finish(g, b):
            av, bv, cv = gbufs[b]
            sa, sb, sc = gsems[b]
            pltpu.make_async_copy(a_hbm.at[sbufs[b]], av, sa).wait()
            pltpu.make_async_copy(b_hbm.at[dstv.at[g]], bv, sb).wait()
            pltpu.make_async_copy(
                c_hbm.at[pl.ds(base_e + (pbase + g) * K, K)], cv, sc).wait()

            shv = jnp.full((16,), 16, jnp.int32)
            mkv = jnp.full((16,), -65536, jnp.int32)
            bc = lambda v: lax.bitcast_convert_type(v, jnp.float32)

            def row(r, _):
                for j in range(D // 32):
                    cw = cv[r, pl.ds(16 * j, 16)]
                    clo = bc(lax.shift_left(cw, shv))
                    chi = bc(lax.bitwise_and(cw, mkv))
                    slo = pl.ds(32 * j, 16)
                    shi = pl.ds(32 * j + 16, 16)
                    av[r, slo] = jnp.maximum(av[r, slo] + bv[r, slo] + clo,
                                             0.0)
                    av[r, shi] = jnp.maximum(av[r, shi] + bv[r, shi] + chi,
                                             0.0)
                return 0

            lax.fori_loop(0, K, row, 0)
            pltpu.sync_copy(av, agg_sh.at[dstv.at[g]], add=True)

        fetch_src(0, 0)
        fetch_src(1, 1)
        issue_gathers(0, 0)

        def pair(p, _):
            g = 2 * p
            issue_gathers(g + 1, 1)
            fetch_src(g + 2, 0)
            finish(g, 0)
            issue_gathers(g + 2, 0)

            @pl.when(g + 3 < PCH)
            def _():
                fetch_src(g + 3, 1)

            finish(g + 1, 1)
            return 0

        lax.fori_loop(0, (PCH - 1) // 2, pair, 0)
        finish(PCH - 1, 0)

    plsc.subcore_barrier()

    r0 = sid * CH
    pltpu.sync_copy(agg_sh.at[pl.ds(r0, CH)], out_hbm.at[cid, pl.ds(r0, CH)])

    @pl.when(sid == 0)
    def _():
        pltpu.sync_copy(agg_sh.at[pl.ds(NS * CH, TAIL)],
                        out_hbm.at[cid, pl.ds(NS * CH, TAIL)])


@functools.cache
def _build_edge_pass():
    return pl.kernel(
        _edge_sc_body,
        out_type=jax.ShapeDtypeStruct((NC, N, D), jnp.float32),
        mesh=plsc.VectorSubcoreMesh(core_axis_name="c", subcore_axis_name="s",
                                    num_cores=NC, num_subcores=NS),
        scratch_types=[
            pltpu.VMEM((PCH, K), jnp.int32),
            pltpu.VMEM((K,), jnp.int32),
            pltpu.VMEM((K,), jnp.int32),
            pltpu.VMEM((K, D), jnp.float32),
            pltpu.VMEM((K, D), jnp.float32),
            pltpu.VMEM((K, D // 2), jnp.int32),
            pltpu.VMEM((K, D), jnp.float32),
            pltpu.VMEM((K, D), jnp.float32),
            pltpu.VMEM((K, D // 2), jnp.int32),
            pltpu.VMEM_SHARED((N, D), jnp.float32),
            pltpu.SemaphoreType.DMA,
            pltpu.SemaphoreType.DMA,
            pltpu.SemaphoreType.DMA,
            pltpu.SemaphoreType.DMA,
            pltpu.SemaphoreType.DMA,
            pltpu.SemaphoreType.DMA,
            pltpu.SemaphoreType.DMA,
            pltpu.SemaphoreType.DMA,
        ],
    )


def _edge_pass(a, b, c, src, dst):
    return _build_edge_pass()(a, b, c, src, dst)


def kernel(node_feat, node_attr, edge_index, edge_attr, batch_index,
           num_sampled_nodes_per_hop, num_sampled_edges_per_hop,
           W_e0, b_e0, W_n0, b_n0, W_e1, b_e1, W_n1, b_n1, W_skip):
    src = edge_index[0].reshape(NW, NCHUNK, K)
    dst = edge_index[1].reshape(NW, NPH, PCH, K)

    Ws0, Wd0, Wr0, Wa0 = (W_e0[:D], W_e0[D:2 * D], W_e0[2 * D:3 * D],
                          W_e0[3 * D:])
    Wsrc0 = Ws0 - Wr0
    Wdst0 = Wd0 + Wr0
    Ws1, Wd1, Wa1 = W_e1[:D], W_e1[D:2 * D], W_e1[2 * D:]

    A0, B0, S = _prep_node(node_feat, Wsrc0, Wdst0, W_skip)
    C0 = _prep_edge(edge_attr, Wa0[:, _Q], b_e0[_Q])

    P0 = _edge_pass(A0, B0, C0, src, dst)
    C1 = _prep_edge(edge_attr, Wa1[:, _Q], b_e1[_Q])
    h1, A1, B1 = _mlp1(node_feat, P0, W_n0[:D], W_n0[D:], b_n0, Ws1, Wd1)

    P1 = _edge_pass(A1, B1, C1, src, dst)
    out = _mlp2(h1, P1, S, W_n1[:D], W_n1[D:], b_n1)

    return (out, node_attr, edge_index, edge_attr)
